# baseline (speedup 1.0000x reference)
import functools

import jax
import jax.numpy as jnp
from jax import lax
from jax.experimental import pallas as pl
from jax.experimental.pallas import tpu as pltpu, tpu_sc as plsc

_NW = 32
_CH = 128
_NB = 4


def _vgather16(x, idx):
    dn = lax.GatherDimensionNumbers(
        offset_dims=(), collapsed_slice_dims=(0,), start_index_map=(0,))
    return lax.gather(x, idx[:, None], dn, (1,),
                      mode=lax.GatherScatterMode.PROMISE_IN_BOUNDS)


def _sc_body(t_total, z2, gt, out, idx_v, data_v, gate_v, gsem, osem):
    nc = 2
    wid = lax.axis_index("s") * nc + lax.axis_index("c")
    rows_per_w = (t_total * 4 * 8) // _NW
    units_per_w = rows_per_w // 64
    u0 = wid * units_per_w
    m = u0 // (t_total // 8)
    ts = (u0 % (t_total // 8)) * 8
    pltpu.sync_copy(gt.at[m, :, pl.ds(ts, units_per_w * 8)], gate_v)
    it16 = lax.broadcasted_iota(jnp.int32, (16,), 0)
    pat_lo = lax.bitwise_and(it16, 7)
    pat_hi = pat_lo + 8
    jpat = lax.shift_right_logical(it16, 3) * 8
    niter = rows_per_w // _CH

    def fire_gather(i):
        buf = i % _NB
        toff = i * 16
        best = gate_v[0, pl.ds(toff, 16)]
        besti = jnp.zeros((16,), jnp.int32)
        for ee in range(1, 8):
            ge = gate_v[ee, pl.ds(toff, 16)]
            gtr = ge > best
            besti = jnp.where(gtr, ee, besti)
            best = jnp.where(gtr, ge, best)
        v16 = m * (t_total * 64) + (ts + toff + it16) * 64 + besti
        dup_lo = _vgather16(v16, pat_lo)
        dup_hi = _vgather16(v16, pat_hi)
        for k in range(8):
            vdup = dup_lo if k < 4 else dup_hi
            idx_v[buf, pl.ds(16 * k, 16)] = vdup + (16 * (k % 4) + jpat)
        pltpu.async_copy(z2.at[idx_v.at[buf]], data_v.at[buf], gsem)

    def wait_gather(i):
        buf = i % _NB
        pltpu.make_async_copy(z2.at[idx_v.at[buf]], data_v.at[buf], gsem).wait()

    def fire_out(i):
        qb = wid * rows_per_w + i * _CH
        pltpu.async_copy(data_v.at[i % _NB], out.at[pl.ds(qb, _CH)], osem)

    def wait_out(i):
        qb = wid * rows_per_w + i * _CH
        pltpu.make_async_copy(data_v.at[i % _NB], out.at[pl.ds(qb, _CH)], osem).wait()

    for p in range(_NB - 1):
        fire_gather(p)
    for i in range(niter):
        if i + _NB - 1 < niter:
            if i >= 1:
                wait_out(i - 1)
            fire_gather(i + _NB - 1)
        wait_gather(i)
        fire_out(i)
    for i in range(max(0, niter - _NB), niter):
        if i >= 1 or niter <= _NB:
            wait_out(i)


def kernel(experts, gate):
    b, t, d, e = experts.shape
    nj = d // 128
    z2 = (experts.transpose(0, 1, 3, 2)
          .reshape(b, t, e, nj, 128)
          .transpose(0, 1, 3, 2, 4)
          .reshape(b * t * nj * e, 128))
    gt = gate.transpose(0, 2, 1)
    mesh = plsc.VectorSubcoreMesh(core_axis_name="c", subcore_axis_name="s")
    run = functools.partial(
        pl.kernel,
        mesh=mesh,
        out_type=jax.ShapeDtypeStruct((b * t * nj, 128), jnp.float32),
        scratch_types=[
            pltpu.VMEM((_NB, _CH), jnp.int32),
            pltpu.VMEM((_NB, _CH, 128), jnp.float32),
            pltpu.VMEM((8, 256), jnp.float32),
            pltpu.SemaphoreType.DMA,
            pltpu.SemaphoreType.DMA,
        ],
    )(functools.partial(_sc_body, t))
    out2 = run(z2, gt)
    return (out2.reshape(b, t // 8, nj, 8, 128)
            .transpose(0, 1, 3, 2, 4)
            .reshape(b, t, d))
